# baseline (speedup 1.0000x reference)
import functools

import jax
import jax.numpy as jnp
from jax import lax
from jax.experimental import pallas as pl
from jax.experimental.pallas import tpu as pltpu
from jax.experimental.pallas import tpu_sc as plsc

N = 10000
E = 320000
D = 128
G = 64
HID = 64

NC = 2
NS = 16
L = 16

EK = 128
NCHUNK = E // EK
NCHP = 2560
NCH32 = NCHUNK // (NC * NS)
NCH16 = NCHP // NS
NB = 2
NGROUPS = NCH16 // NB
RPT = 640
RPT_LAST = N - (NS - 1) * RPT
CH = 40

_mesh = plsc.VectorSubcoreMesh(core_axis_name="c", subcore_axis_name="s")


def _deg_body(edge3, zeros, ones8, deg_out, dstbuf, ones_v, deg_s, sem_i, sem_s):
    c = lax.axis_index("c")
    s = lax.axis_index("s")
    r0 = s * RPT
    w = c * NS + s

    pltpu.sync_copy(ones8, ones_v)

    @pl.when(s < NS - 1)
    def _():
        pltpu.sync_copy(zeros.at[pl.ds(r0, RPT), pl.ds(0, 8)],
                        deg_s.at[pl.ds(r0, RPT)])

    @pl.when(s == NS - 1)
    def _():
        pltpu.sync_copy(zeros.at[pl.ds(r0, RPT_LAST), pl.ds(0, 8)],
                        deg_s.at[pl.ds(r0, RPT_LAST)])

    plsc.subcore_barrier()

    ch0 = w * NCH32

    def fire_idx(j, p):
        pltpu.async_copy(edge3.at[1, pl.ds(ch0 + j, 1), :], dstbuf.at[p],
                         sem_i)

    def drain_idx(p):
        pltpu.make_async_copy(edge3.at[1, pl.ds(0, 1), :], dstbuf.at[p],
                              sem_i).wait()

    fire_idx(0, 0)

    def step2(t, carry):
        j0 = 2 * t
        drain_idx(0)
        fire_idx(j0 + 1, 1)
        pltpu.async_copy(ones_v, deg_s.at[dstbuf.at[0, 0]], sem_s, add=True)
        drain_idx(1)

        @pl.when(t < NCH32 // 2 - 1)
        def _():
            fire_idx(j0 + 2, 0)

        pltpu.async_copy(ones_v, deg_s.at[dstbuf.at[1, 0]], sem_s, add=True)
        return carry

    lax.fori_loop(0, NCH32 // 2, step2, 0)

    @pl.when(w < NCHUNK - NC * NS * NCH32)
    def _():
        pltpu.sync_copy(edge3.at[1, pl.ds(NC * NS * NCH32 + w, 1), :],
                        dstbuf.at[0])
        pltpu.async_copy(ones_v, deg_s.at[dstbuf.at[0, 0]], sem_s, add=True)

    def drain_s(j, carry):
        pltpu.make_async_copy(zeros.at[pl.ds(0, EK), pl.ds(0, 8)],
                              ones_v, sem_s).wait()
        return carry

    nsc = NCH32 + jnp.where(w < NCHUNK - NC * NS * NCH32, 1, 0)
    lax.fori_loop(0, nsc, drain_s, 0)
    plsc.subcore_barrier()

    @pl.when(s < NS - 1)
    def _():
        pltpu.sync_copy(deg_s.at[pl.ds(r0, RPT), pl.ds(0, 8)],
                        deg_out.at[c, pl.ds(r0, RPT), :])

    @pl.when(s == NS - 1)
    def _():
        pltpu.sync_copy(deg_s.at[pl.ds(r0, RPT_LAST), pl.ds(0, 8)],
                        deg_out.at[c, pl.ds(r0, RPT_LAST), :])


_deg_kernel = functools.partial(
    pl.kernel,
    out_type=jax.ShapeDtypeStruct((NC, N, 8), jnp.float32),
    mesh=_mesh,
    scratch_types=[
        pltpu.VMEM((2, 1, EK), jnp.int32),
        pltpu.VMEM((EK, 8), jnp.float32),
        pltpu.VMEM_SHARED((N + 8, 8), jnp.float32),
        pltpu.SemaphoreType.DMA,
        pltpu.SemaphoreType.DMA,
    ],
    compiler_params=pltpu.CompilerParams(use_tc_tiling_on_sc=False),
)(_deg_body)


def _msg_body(layer2, dstp, offs, gs, dis, b, batch, zeros, out,
              offbuf, didx, rows, abuf, gbuf, ybuf, disbuf, bbuf,
              batchbuf, batchoff, acc_s, pooled_s, sem_i, sem_g, sem_s):
    npasses = 2 if layer2 else 1
    HW = 64
    KB = HW // L
    c = lax.axis_index("c")
    s = lax.axis_index("s")
    r0 = s * RPT
    ch0 = s * NCH16

    if layer2:
        @pl.when(s == 0)
        def _():
            pltpu.sync_copy(zeros.at[pl.ds(0, 2 * G), pl.ds(0, HW)], pooled_s)

    for q in range(npasses):
        idxrow = npasses * c + q
        qoff = idxrow * N

        @pl.when(s < NS - 1)
        def _():
            pltpu.sync_copy(zeros.at[pl.ds(r0, RPT), pl.ds(0, HW)],
                            acc_s.at[pl.ds(r0, RPT)])

        @pl.when(s == NS - 1)
        def _():
            pltpu.sync_copy(zeros.at[pl.ds(r0, RPT_LAST), pl.ds(0, HW)],
                            acc_s.at[pl.ds(r0, RPT_LAST)])

        pltpu.sync_copy(b.at[pl.ds(idxrow * HW, HW)], bbuf)
        if layer2:
            z16f = jnp.zeros((L,), jnp.float32)
            for i in range(CH, 48):
                for k in range(4):
                    ybuf[i, pl.ds(k * L, L)] = z16f
            batchbuf[pl.ds(32, L)] = jnp.zeros((L,), jnp.int32)

        plsc.subcore_barrier()

        def fire_idx(gi, p):
            cb = ch0 + gi * NB
            pltpu.async_copy(offs.at[idxrow, pl.ds(cb, NB), :], offbuf.at[p],
                             sem_i)
            pltpu.async_copy(dstp.at[pl.ds(cb, NB), :], didx.at[p], sem_i)

        def drain_idx(p):
            pltpu.make_async_copy(dstp.at[pl.ds(0, NB), :], offbuf.at[p],
                                  sem_i).wait()
            pltpu.make_async_copy(dstp.at[pl.ds(0, NB), :], didx.at[p],
                                  sem_i).wait()

        def drain_scatters(p):
            for bb in range(NB):
                pltpu.make_async_copy(gs.at[pl.ds(0, EK), :], rows.at[p, bb],
                                      sem_s).wait()

        def phase(g, p, drain_sc, fire_next):
            drain_idx(p)
            gds = [pltpu.async_copy(gs.at[offbuf.at[p, bb]], rows.at[p, bb],
                                    sem_g) for bb in range(NB)]
            for d in gds:
                d.wait()
            drain_sc()
            fire_next()
            for bb in range(NB):
                pltpu.async_copy(rows.at[p, bb], acc_s.at[didx.at[p, bb]],
                                 sem_s, add=True)

        fire_idx(0, 0)

        def pair(t, carry):
            g0 = 2 * t

            def drain0():
                @pl.when(t > 0)
                def _():
                    drain_scatters(1)

            phase(g0, 0, drain0, lambda: fire_idx(g0 + 1, 1))

            def fire1():
                @pl.when(t < NGROUPS // 2 - 1)
                def _():
                    fire_idx(g0 + 2, 0)

            phase(g0 + 1, 1, lambda: drain_scatters(0), fire1)
            return carry

        lax.fori_loop(0, NGROUPS // 2, pair, 0)
        drain_scatters(1)
        plsc.subcore_barrier()

        def chunk(j, carry):
            row = r0 + j * CH
            pltpu.sync_copy(acc_s.at[pl.ds(row, CH)], abuf)
            pltpu.sync_copy(gs.at[pl.ds(qoff + row, CH), :], gbuf)
            pltpu.sync_copy(dis.at[pl.ds(row, CH)], disbuf.at[pl.ds(0, CH)])
            dvs = [disbuf[pl.ds(k2 * L, L)] for k2 in range(3)]
            for i in range(CH):
                d = dvs[i // L][i % L]
                for k in range(KB):
                    sl = pl.ds(k * L, L)
                    t = (abuf[i, sl] + gbuf[i, sl]) * d + bbuf[sl]
                    if not layer2:
                        t = jnp.maximum(t, 0.0)
                    ybuf[i, sl] = t
            if layer2:
                pltpu.sync_copy(batch.at[pl.ds(row, CH)], batchbuf.at[pl.ds(0, CH)])
                for k2 in range(3):
                    sl = pl.ds(k2 * L, L)
                    batchoff[sl] = batchbuf[sl] + q * G
                pltpu.sync_copy(ybuf, pooled_s.at[batchoff], add=True)
            else:
                pltpu.sync_copy(ybuf.at[pl.ds(0, CH)],
                                out.at[pl.ds(row, CH), pl.ds(c * HW, HW)])
            return carry

        @pl.when(s < NS - 1)
        def _():
            lax.fori_loop(0, RPT // CH, chunk, 0)

        @pl.when(s == NS - 1)
        def _():
            lax.fori_loop(0, RPT_LAST // CH, chunk, 0)

        plsc.subcore_barrier()

    if layer2:
        @pl.when(s == 0)
        def _():
            for q in range(2):
                pltpu.sync_copy(
                    pooled_s.at[pl.ds(q * G, G), :],
                    out.at[:, pl.ds((2 * c + q) * HW, HW)])


def _make_msg_kernel(layer2):
    HW = 64
    out_ty = (jax.ShapeDtypeStruct((G, 256), jnp.float32) if layer2
              else jax.ShapeDtypeStruct((N, 128), jnp.float32))
    return functools.partial(
        pl.kernel,
        out_type=out_ty,
        mesh=_mesh,
        scratch_types=[
            pltpu.VMEM((2, NB, EK), jnp.int32),
            pltpu.VMEM((2, NB, EK), jnp.int32),
            pltpu.VMEM((2, NB, EK, 64), jnp.float32),
            pltpu.VMEM((CH, 64), jnp.float32),
            pltpu.VMEM((CH, 64), jnp.float32),
            pltpu.VMEM((48, 64), jnp.float32),
            pltpu.VMEM((48,), jnp.float32),
            pltpu.VMEM((64,), jnp.float32),
            pltpu.VMEM((48,), jnp.int32),
            pltpu.VMEM((48,), jnp.int32),
            pltpu.VMEM_SHARED((N + 8, 64), jnp.float32),
            pltpu.VMEM_SHARED((2 * G, 64), jnp.float32),
            pltpu.SemaphoreType.DMA,
            pltpu.SemaphoreType.DMA,
            pltpu.SemaphoreType.DMA,
        ],
        compiler_params=pltpu.CompilerParams(use_tc_tiling_on_sc=False),
    )(functools.partial(_msg_body, layer2))


_msg1_kernel = _make_msg_kernel(False)
_msg2_kernel = _make_msg_kernel(True)


def _lin1_body(x_ref, w_ref, deg2_ref, batch_ref, esrc_ref, edst_ref, gs_ref, dis_ref, cnt_ref, offs_ref, dstp_ref):
    deg = deg2_ref[0, :, 0] + deg2_ref[1, :, 0] + 1.0
    dis = lax.rsqrt(deg)
    dis_ref[...] = dis
    h = jnp.dot(x_ref[...], w_ref[...], preferred_element_type=jnp.float32)
    g = h * dis[:, None]
    gs_ref[0:N, :] = g[:, 0:64]
    gs_ref[N:2 * N, :] = g[:, 64:128]
    b = batch_ref[...]
    ids = lax.broadcasted_iota(jnp.int32, (G, N), 0)
    cnt_ref[...] = jnp.sum(
        jnp.where(b[None, :] == ids, 1.0, 0.0), axis=1, keepdims=True)
    src = esrc_ref[...]
    for k in range(4):
        offs_ref[k, 0:NCHUNK, :] = src + k * N
        offs_ref[k, NCHUNK:NCHP, :] = jnp.full(
            (NCHP - NCHUNK, EK), k * N, jnp.int32)
    dstp_ref[0:NCHUNK, :] = edst_ref[...]
    dstp_ref[NCHUNK:NCHP, :] = jnp.full((NCHP - NCHUNK, EK), N, jnp.int32)


def _lin2_body(y_ref, w_ref, dis_ref, gs_ref):
    h = jnp.dot(y_ref[...], w_ref[...], preferred_element_type=jnp.float32)
    g = h * dis_ref[...][:, None]
    for q in range(4):
        gs_ref[q * N:(q + 1) * N, :] = g[:, q * 64:(q + 1) * 64]


def _head_body(pooled_ref, cnt_ref, w_ref, b_ref, out_ref):
    pooled = pooled_ref[...] / jnp.maximum(cnt_ref[...], 1.0)
    out_ref[...] = (
        jnp.dot(pooled, w_ref[...], preferred_element_type=jnp.float32)
        + b_ref[...][None, :])


def kernel(x, edge_index, batch, W1, b1, W2, b2, Wfc, bfc):
    zeros = jnp.zeros((N, 128), jnp.float32)
    edge3 = edge_index.reshape(2, NCHUNK, EK)

    ones8 = jnp.ones((EK, 8), jnp.float32)
    deg2 = _deg_kernel(edge3, zeros, ones8)

    gs1, dis, cnt, offs, dstp = pl.pallas_call(
        _lin1_body,
        out_shape=[
            jax.ShapeDtypeStruct((2 * N, 64), jnp.float32),
            jax.ShapeDtypeStruct((N,), jnp.float32),
            jax.ShapeDtypeStruct((G, 1), jnp.float32),
            jax.ShapeDtypeStruct((4, NCHP, EK), jnp.int32),
            jax.ShapeDtypeStruct((NCHP, EK), jnp.int32),
        ],
    )(x, W1, deg2, batch, edge3[0], edge3[1])

    y1 = _msg1_kernel(dstp, offs, gs1, dis, b1, batch, zeros)

    gs2 = pl.pallas_call(
        _lin2_body,
        out_shape=jax.ShapeDtypeStruct((4 * N, 64), jnp.float32),
    )(y1, W2, dis)

    pooled = _msg2_kernel(dstp, offs, gs2, dis, b2, batch, zeros)

    out = pl.pallas_call(
        _head_body,
        out_shape=jax.ShapeDtypeStruct((G, HID), jnp.float32),
    )(pooled, cnt, Wfc, bfc)
    return out


# speedup vs baseline: 2.0836x; 2.0836x over previous
import functools

import jax
import jax.numpy as jnp
from jax import lax
from jax.experimental import pallas as pl
from jax.experimental.pallas import tpu as pltpu
from jax.experimental.pallas import tpu_sc as plsc

N = 10000
E = 320000
D = 128
G = 64
HID = 64

NC = 2
NS = 16
L = 16

EK = 80
NCHUNK = E // EK
NCHP = NCHUNK
NCH32 = NCHUNK // (NC * NS)
NCH16 = NCHP // NS
NB = 5
NGROUPS = NCH16 // NB
RPT = 640
RPT_LAST = N - (NS - 1) * RPT
CH = 40

_mesh = plsc.VectorSubcoreMesh(core_axis_name="c", subcore_axis_name="s")


def _deg_body(edge3, zeros, ones8, deg_out, dstbuf, ones_v, deg_s, sem_i, sem_s):
    c = lax.axis_index("c")
    s = lax.axis_index("s")
    r0 = s * RPT
    w = c * NS + s

    pltpu.sync_copy(ones8, ones_v)

    @pl.when(s < NS - 1)
    def _():
        pltpu.sync_copy(zeros.at[pl.ds(r0, RPT), pl.ds(0, 8)],
                        deg_s.at[pl.ds(r0, RPT)])

    @pl.when(s == NS - 1)
    def _():
        pltpu.sync_copy(zeros.at[pl.ds(r0, RPT_LAST), pl.ds(0, 8)],
                        deg_s.at[pl.ds(r0, RPT_LAST)])

    plsc.subcore_barrier()

    ch0 = w * NCH32

    def fire_idx(j, p):
        pltpu.async_copy(edge3.at[1, pl.ds(ch0 + j, 1), :], dstbuf.at[p],
                         sem_i)

    def drain_idx(p):
        pltpu.make_async_copy(edge3.at[1, pl.ds(0, 1), :], dstbuf.at[p],
                              sem_i).wait()

    fire_idx(0, 0)

    def step2(t, carry):
        j0 = 2 * t
        drain_idx(0)
        fire_idx(j0 + 1, 1)
        pltpu.async_copy(ones_v, deg_s.at[dstbuf.at[0, 0]], sem_s, add=True)
        drain_idx(1)

        @pl.when(t < GUARD)
        def _():
            fire_idx(j0 + 2, 0)

        pltpu.async_copy(ones_v, deg_s.at[dstbuf.at[1, 0]], sem_s, add=True)
        return carry

    GUARD = NCH32 // 2 - 1 if NCH32 % 2 == 0 else NCH32 // 2
    lax.fori_loop(0, NCH32 // 2, step2, 0)
    if NCH32 % 2 == 1:
        drain_idx(0)
        pltpu.async_copy(ones_v, deg_s.at[dstbuf.at[0, 0]], sem_s, add=True)

    def drain_s(j, carry):
        pltpu.make_async_copy(zeros.at[pl.ds(0, EK), pl.ds(0, 8)],
                              ones_v, sem_s).wait()
        return carry

    lax.fori_loop(0, NCH32, drain_s, 0)
    plsc.subcore_barrier()

    @pl.when(s < NS - 1)
    def _():
        pltpu.sync_copy(deg_s.at[pl.ds(r0, RPT), pl.ds(0, 8)],
                        deg_out.at[c, pl.ds(r0, RPT), :])

    @pl.when(s == NS - 1)
    def _():
        pltpu.sync_copy(deg_s.at[pl.ds(r0, RPT_LAST), pl.ds(0, 8)],
                        deg_out.at[c, pl.ds(r0, RPT_LAST), :])


_deg_kernel = functools.partial(
    pl.kernel,
    out_type=jax.ShapeDtypeStruct((NC, N, 8), jnp.float32),
    mesh=_mesh,
    scratch_types=[
        pltpu.VMEM((2, 1, EK), jnp.int32),
        pltpu.VMEM((EK, 8), jnp.float32),
        pltpu.VMEM_SHARED((N + 8, 8), jnp.float32),
        pltpu.SemaphoreType.DMA,
        pltpu.SemaphoreType.DMA,
    ],
    compiler_params=pltpu.CompilerParams(use_tc_tiling_on_sc=False),
)(_deg_body)


def _msg_body(layer2, dstp, offs, gs, dis, b, batch, zeros, out,
              offbuf, didx, rows, abuf, gbuf, ybuf, disbuf, bbuf,
              batchbuf, batchoff, acc_s, pooled_s, sem_i, sem_g, sem_s):
    npasses = 2 if layer2 else 1
    HW = 64
    KB = HW // L
    c = lax.axis_index("c")
    s = lax.axis_index("s")
    r0 = s * RPT
    ch0 = s * NCH16

    if layer2:
        @pl.when(s == 0)
        def _():
            pltpu.sync_copy(zeros.at[pl.ds(0, 2 * G), pl.ds(0, HW)], pooled_s)

    for q in range(npasses):
        idxrow = npasses * c + q
        qoff = idxrow * N

        @pl.when(s < NS - 1)
        def _():
            pltpu.sync_copy(zeros.at[pl.ds(r0, RPT), pl.ds(0, HW)],
                            acc_s.at[pl.ds(r0, RPT)])

        @pl.when(s == NS - 1)
        def _():
            pltpu.sync_copy(zeros.at[pl.ds(r0, RPT_LAST), pl.ds(0, HW)],
                            acc_s.at[pl.ds(r0, RPT_LAST)])

        pltpu.sync_copy(b.at[pl.ds(idxrow * HW, HW)], bbuf)
        if layer2:
            z16f = jnp.zeros((L,), jnp.float32)
            for i in range(CH, 48):
                for k in range(4):
                    ybuf[i, pl.ds(k * L, L)] = z16f
            batchbuf[pl.ds(32, L)] = jnp.zeros((L,), jnp.int32)

        plsc.subcore_barrier()

        def fire_idx(gi, p):
            cb = ch0 + gi * NB
            pltpu.async_copy(offs.at[idxrow, pl.ds(cb, NB), :], offbuf.at[p],
                             sem_i)
            pltpu.async_copy(dstp.at[pl.ds(cb, NB), :], didx.at[p], sem_i)

        def drain_idx(p):
            pltpu.make_async_copy(dstp.at[pl.ds(0, NB), :], offbuf.at[p],
                                  sem_i).wait()
            pltpu.make_async_copy(dstp.at[pl.ds(0, NB), :], didx.at[p],
                                  sem_i).wait()

        def drain_scatters(p):
            for bb in range(NB):
                pltpu.make_async_copy(gs.at[pl.ds(0, EK), :], rows.at[p, bb],
                                      sem_s).wait()

        def phase(g, p, drain_sc, fire_next):
            drain_idx(p)
            gds = [pltpu.async_copy(gs.at[offbuf.at[p, bb]], rows.at[p, bb],
                                    sem_g) for bb in range(NB)]
            for d in gds:
                d.wait()
            drain_sc()
            fire_next()
            for bb in range(NB):
                pltpu.async_copy(rows.at[p, bb], acc_s.at[didx.at[p, bb]],
                                 sem_s, add=True)

        fire_idx(0, 0)

        def pair(t, carry):
            g0 = 2 * t

            def drain0():
                @pl.when(t > 0)
                def _():
                    drain_scatters(1)

            phase(g0, 0, drain0, lambda: fire_idx(g0 + 1, 1))

            def fire1():
                @pl.when(t < NGROUPS // 2 - 1)
                def _():
                    fire_idx(g0 + 2, 0)

            phase(g0 + 1, 1, lambda: drain_scatters(0), fire1)
            return carry

        lax.fori_loop(0, NGROUPS // 2, pair, 0)
        drain_scatters(1)
        plsc.subcore_barrier()

        def chunk(j, carry):
            row = r0 + j * CH
            pltpu.sync_copy(acc_s.at[pl.ds(row, CH)], abuf)
            pltpu.sync_copy(gs.at[pl.ds(qoff + row, CH), :], gbuf)
            pltpu.sync_copy(dis.at[pl.ds(row, CH)], disbuf.at[pl.ds(0, CH)])
            dvs = [disbuf[pl.ds(k2 * L, L)] for k2 in range(3)]
            for i in range(CH):
                d = dvs[i // L][i % L]
                for k in range(KB):
                    sl = pl.ds(k * L, L)
                    t = (abuf[i, sl] + gbuf[i, sl]) * d + bbuf[sl]
                    if not layer2:
                        t = jnp.maximum(t, 0.0)
                    ybuf[i, sl] = t
            if layer2:
                pltpu.sync_copy(batch.at[pl.ds(row, CH)], batchbuf.at[pl.ds(0, CH)])
                for k2 in range(3):
                    sl = pl.ds(k2 * L, L)
                    batchoff[sl] = batchbuf[sl] + q * G
                pltpu.sync_copy(ybuf, pooled_s.at[batchoff], add=True)
            else:
                pltpu.sync_copy(ybuf.at[pl.ds(0, CH)],
                                out.at[pl.ds(row, CH), pl.ds(c * HW, HW)])
            return carry

        @pl.when(s < NS - 1)
        def _():
            lax.fori_loop(0, RPT // CH, chunk, 0)

        @pl.when(s == NS - 1)
        def _():
            lax.fori_loop(0, RPT_LAST // CH, chunk, 0)

        plsc.subcore_barrier()

    if layer2:
        @pl.when(s == 0)
        def _():
            for q in range(2):
                pltpu.sync_copy(
                    pooled_s.at[pl.ds(q * G, G), :],
                    out.at[:, pl.ds((2 * c + q) * HW, HW)])


def _make_msg_kernel(layer2):
    HW = 64
    out_ty = (jax.ShapeDtypeStruct((G, 256), jnp.float32) if layer2
              else jax.ShapeDtypeStruct((N, 128), jnp.float32))
    return functools.partial(
        pl.kernel,
        out_type=out_ty,
        mesh=_mesh,
        scratch_types=[
            pltpu.VMEM((2, NB, EK), jnp.int32),
            pltpu.VMEM((2, NB, EK), jnp.int32),
            pltpu.VMEM((2, NB, EK, 64), jnp.float32),
            pltpu.VMEM((CH, 64), jnp.float32),
            pltpu.VMEM((CH, 64), jnp.float32),
            pltpu.VMEM((48, 64), jnp.float32),
            pltpu.VMEM((48,), jnp.float32),
            pltpu.VMEM((64,), jnp.float32),
            pltpu.VMEM((48,), jnp.int32),
            pltpu.VMEM((48,), jnp.int32),
            pltpu.VMEM_SHARED((N + 8, 64), jnp.float32),
            pltpu.VMEM_SHARED((2 * G, 64), jnp.float32),
            pltpu.SemaphoreType.DMA,
            pltpu.SemaphoreType.DMA,
            pltpu.SemaphoreType.DMA,
        ],
        compiler_params=pltpu.CompilerParams(use_tc_tiling_on_sc=False),
    )(functools.partial(_msg_body, layer2))


_msg1_kernel = _make_msg_kernel(False)
_msg2_kernel = _make_msg_kernel(True)


def _lin1_body(x_ref, w_ref, deg2_ref, batch_ref, esrc_ref, edst_ref, gs_ref, dis_ref, cnt_ref, offs_ref, dstp_ref):
    deg = deg2_ref[0, :, 0] + deg2_ref[1, :, 0] + 1.0
    dis = lax.rsqrt(deg)
    dis_ref[...] = dis
    h = jnp.dot(x_ref[...], w_ref[...], preferred_element_type=jnp.float32)
    g = h * dis[:, None]
    gs_ref[0:N, :] = g[:, 0:64]
    gs_ref[N:2 * N, :] = g[:, 64:128]
    b = batch_ref[...]
    ids = lax.broadcasted_iota(jnp.int32, (G, N), 0)
    cnt_ref[...] = jnp.sum(
        jnp.where(b[None, :] == ids, 1.0, 0.0), axis=1, keepdims=True)
    src = esrc_ref[...]
    for k in range(4):
        offs_ref[k, 0:NCHUNK, :] = src + k * N
        if NCHP > NCHUNK:
            offs_ref[k, NCHUNK:NCHP, :] = jnp.full(
                (NCHP - NCHUNK, EK), k * N, jnp.int32)
    dstp_ref[0:NCHUNK, :] = edst_ref[...]
    if NCHP > NCHUNK:
        dstp_ref[NCHUNK:NCHP, :] = jnp.full((NCHP - NCHUNK, EK), N, jnp.int32)


def _lin2_body(y_ref, w_ref, dis_ref, gs_ref):
    h = jnp.dot(y_ref[...], w_ref[...], preferred_element_type=jnp.float32)
    g = h * dis_ref[...][:, None]
    for q in range(4):
        gs_ref[q * N:(q + 1) * N, :] = g[:, q * 64:(q + 1) * 64]


def _head_body(pooled_ref, cnt_ref, w_ref, b_ref, out_ref):
    pooled = pooled_ref[...] / jnp.maximum(cnt_ref[...], 1.0)
    out_ref[...] = (
        jnp.dot(pooled, w_ref[...], preferred_element_type=jnp.float32)
        + b_ref[...][None, :])


def kernel(x, edge_index, batch, W1, b1, W2, b2, Wfc, bfc):
    zeros = jnp.zeros((N, 128), jnp.float32)
    edge3 = edge_index.reshape(2, NCHUNK, EK)

    ones8 = jnp.ones((EK, 8), jnp.float32)
    deg2 = _deg_kernel(edge3, zeros, ones8)

    gs1, dis, cnt, offs, dstp = pl.pallas_call(
        _lin1_body,
        out_shape=[
            jax.ShapeDtypeStruct((2 * N, 64), jnp.float32),
            jax.ShapeDtypeStruct((N,), jnp.float32),
            jax.ShapeDtypeStruct((G, 1), jnp.float32),
            jax.ShapeDtypeStruct((4, NCHP, EK), jnp.int32),
            jax.ShapeDtypeStruct((NCHP, EK), jnp.int32),
        ],
    )(x, W1, deg2, batch, edge3[0], edge3[1])

    y1 = _msg1_kernel(dstp, offs, gs1, dis, b1, batch, zeros)

    gs2 = pl.pallas_call(
        _lin2_body,
        out_shape=jax.ShapeDtypeStruct((4 * N, 64), jnp.float32),
    )(y1, W2, dis)

    pooled = _msg2_kernel(dstp, offs, gs2, dis, b2, batch, zeros)

    out = pl.pallas_call(
        _head_body,
        out_shape=jax.ShapeDtypeStruct((G, HID), jnp.float32),
    )(pooled, cnt, Wfc, bfc)
    return out


# speedup vs baseline: 2.1184x; 1.0167x over previous
import functools

import jax
import jax.numpy as jnp
from jax import lax
from jax.experimental import pallas as pl
from jax.experimental.pallas import tpu as pltpu
from jax.experimental.pallas import tpu_sc as plsc

N = 10000
E = 320000
D = 128
G = 64
HID = 64

NC = 2
NS = 16
L = 16

EK = 80
NCHUNK = E // EK
NCHP = NCHUNK
NCH32 = NCHUNK // (NC * NS)
NCH16 = NCHP // NS
NB = 5
NGROUPS = NCH16 // NB
RPT = 640
RPT_LAST = N - (NS - 1) * RPT
CH = 40

_mesh = plsc.VectorSubcoreMesh(core_axis_name="c", subcore_axis_name="s")


def _deg_body(edge3, zeros, ones8, deg_out, dstbuf, ones_v, deg_s, sem_i, sem_s):
    c = lax.axis_index("c")
    s = lax.axis_index("s")
    r0 = s * RPT
    w = c * NS + s

    pltpu.sync_copy(ones8, ones_v)

    @pl.when(s < NS - 1)
    def _():
        pltpu.sync_copy(zeros.at[pl.ds(r0, RPT), pl.ds(0, 8)],
                        deg_s.at[pl.ds(r0, RPT)])

    @pl.when(s == NS - 1)
    def _():
        pltpu.sync_copy(zeros.at[pl.ds(r0, RPT_LAST), pl.ds(0, 8)],
                        deg_s.at[pl.ds(r0, RPT_LAST)])

    plsc.subcore_barrier()

    ch0 = w * NCH32

    def fire_idx(j, p):
        pltpu.async_copy(edge3.at[1, pl.ds(ch0 + j, 1), :], dstbuf.at[p],
                         sem_i)

    def drain_idx(p):
        pltpu.make_async_copy(edge3.at[1, pl.ds(0, 1), :], dstbuf.at[p],
                              sem_i).wait()

    fire_idx(0, 0)

    def step2(t, carry):
        j0 = 2 * t
        drain_idx(0)
        fire_idx(j0 + 1, 1)
        pltpu.async_copy(ones_v, deg_s.at[dstbuf.at[0, 0]], sem_s, add=True)
        drain_idx(1)

        @pl.when(t < GUARD)
        def _():
            fire_idx(j0 + 2, 0)

        pltpu.async_copy(ones_v, deg_s.at[dstbuf.at[1, 0]], sem_s, add=True)
        return carry

    GUARD = NCH32 // 2 - 1 if NCH32 % 2 == 0 else NCH32 // 2
    lax.fori_loop(0, NCH32 // 2, step2, 0)
    if NCH32 % 2 == 1:
        drain_idx(0)
        pltpu.async_copy(ones_v, deg_s.at[dstbuf.at[0, 0]], sem_s, add=True)

    def drain_s(j, carry):
        pltpu.make_async_copy(zeros.at[pl.ds(0, EK), pl.ds(0, 8)],
                              ones_v, sem_s).wait()
        return carry

    lax.fori_loop(0, NCH32, drain_s, 0)
    plsc.subcore_barrier()

    @pl.when(s < NS - 1)
    def _():
        pltpu.sync_copy(deg_s.at[pl.ds(r0, RPT), pl.ds(0, 8)],
                        deg_out.at[c, pl.ds(r0, RPT), :])

    @pl.when(s == NS - 1)
    def _():
        pltpu.sync_copy(deg_s.at[pl.ds(r0, RPT_LAST), pl.ds(0, 8)],
                        deg_out.at[c, pl.ds(r0, RPT_LAST), :])


_deg_kernel = functools.partial(
    pl.kernel,
    out_type=jax.ShapeDtypeStruct((NC, N, 8), jnp.float32),
    mesh=_mesh,
    scratch_types=[
        pltpu.VMEM((2, 1, EK), jnp.int32),
        pltpu.VMEM((EK, 8), jnp.float32),
        pltpu.VMEM_SHARED((N + 8, 8), jnp.float32),
        pltpu.SemaphoreType.DMA,
        pltpu.SemaphoreType.DMA,
    ],
    compiler_params=pltpu.CompilerParams(use_tc_tiling_on_sc=False),
)(_deg_body)


def _msg_body(layer2, dstp, offs, gs, dis, b, batch, zeros, out,
              offbuf, didx, rows, abuf, gbuf, ybuf, disbuf, bbuf,
              batchbuf, batchoff, acc_s, pooled_s, sem_i, sem_g, sem_s):
    npasses = 2 if layer2 else 1
    HW = 64
    KB = HW // L
    c = lax.axis_index("c")
    s = lax.axis_index("s")
    r0 = s * RPT
    ch0 = s * NCH16

    if layer2:
        @pl.when(s == 0)
        def _():
            pltpu.sync_copy(zeros.at[pl.ds(0, 2 * G), pl.ds(0, HW)], pooled_s)

    for q in range(npasses):
        idxrow = npasses * c + q
        qoff = idxrow * N

        @pl.when(s < NS - 1)
        def _():
            pltpu.sync_copy(zeros.at[pl.ds(r0, RPT), pl.ds(0, HW)],
                            acc_s.at[pl.ds(r0, RPT)])

        @pl.when(s == NS - 1)
        def _():
            pltpu.sync_copy(zeros.at[pl.ds(r0, RPT_LAST), pl.ds(0, HW)],
                            acc_s.at[pl.ds(r0, RPT_LAST)])

        pltpu.sync_copy(b.at[pl.ds(idxrow * HW, HW)], bbuf)
        if layer2:
            z16f = jnp.zeros((L,), jnp.float32)
            for i in range(CH, 48):
                for k in range(4):
                    ybuf[i, pl.ds(k * L, L)] = z16f
            batchbuf[pl.ds(32, L)] = jnp.zeros((L,), jnp.int32)

        plsc.subcore_barrier()

        def fire_idx(gi, p):
            cb = ch0 + gi * NB
            pltpu.async_copy(offs.at[idxrow, pl.ds(cb, NB), :], offbuf.at[p],
                             sem_i)
            pltpu.async_copy(dstp.at[pl.ds(cb, NB), :], didx.at[p], sem_i)

        def drain_idx(p):
            pltpu.make_async_copy(dstp.at[pl.ds(0, NB), :], offbuf.at[p],
                                  sem_i).wait()
            pltpu.make_async_copy(dstp.at[pl.ds(0, NB), :], didx.at[p],
                                  sem_i).wait()

        def drain_scatters(p):
            for bb in range(NB):
                pltpu.make_async_copy(gs.at[pl.ds(0, EK), :], rows.at[p, bb],
                                      sem_s).wait()

        def phase(g, p, drain_sc, fire_next):
            drain_idx(p)
            gds = [pltpu.async_copy(gs.at[offbuf.at[p, bb]], rows.at[p, bb],
                                    sem_g) for bb in range(NB)]
            for d in gds:
                d.wait()
            drain_sc()
            fire_next()
            for bb in range(NB):
                pltpu.async_copy(rows.at[p, bb], acc_s.at[didx.at[p, bb]],
                                 sem_s, add=True)

        fire_idx(0, 0)

        def pair(t, carry):
            g0 = 2 * t

            def drain0():
                @pl.when(t > 0)
                def _():
                    drain_scatters(1)

            phase(g0, 0, drain0, lambda: fire_idx(g0 + 1, 1))

            def fire1():
                @pl.when(t < NGROUPS // 2 - 1)
                def _():
                    fire_idx(g0 + 2, 0)

            phase(g0 + 1, 1, lambda: drain_scatters(0), fire1)
            return carry

        lax.fori_loop(0, NGROUPS // 2, pair, 0)
        drain_scatters(1)
        plsc.subcore_barrier()

        def chunk(j, carry):
            row = r0 + j * CH
            pltpu.sync_copy(acc_s.at[pl.ds(row, CH)], abuf)
            pltpu.sync_copy(gs.at[pl.ds(qoff + row, CH), :], gbuf)
            pltpu.sync_copy(dis.at[pl.ds(row, CH)], disbuf.at[pl.ds(0, CH)])
            dvs = [disbuf[pl.ds(k2 * L, L)] for k2 in range(3)]
            for i in range(CH):
                d = dvs[i // L][i % L]
                for k in range(KB):
                    sl = pl.ds(k * L, L)
                    t = (abuf[i, sl] + gbuf[i, sl]) * d + bbuf[sl]
                    if not layer2:
                        t = jnp.maximum(t, 0.0)
                    ybuf[i, sl] = t
            if layer2:
                pltpu.sync_copy(batch.at[pl.ds(row, CH)], batchbuf.at[pl.ds(0, CH)])
                for k2 in range(3):
                    sl = pl.ds(k2 * L, L)
                    batchoff[sl] = batchbuf[sl] + q * G
                pltpu.sync_copy(ybuf, pooled_s.at[batchoff], add=True)
            else:
                pltpu.sync_copy(ybuf.at[pl.ds(0, CH)],
                                out.at[pl.ds(row, CH), pl.ds(c * HW, HW)])
            return carry

        @pl.when(s < NS - 1)
        def _():
            lax.fori_loop(0, RPT // CH, chunk, 0)

        @pl.when(s == NS - 1)
        def _():
            lax.fori_loop(0, RPT_LAST // CH, chunk, 0)

        plsc.subcore_barrier()

    if layer2:
        @pl.when(s == 0)
        def _():
            for q in range(2):
                pltpu.sync_copy(
                    pooled_s.at[pl.ds(q * G, G), :],
                    out.at[:, pl.ds((2 * c + q) * HW, HW)])


def _make_msg_kernel(layer2):
    HW = 64
    out_ty = (jax.ShapeDtypeStruct((G, 256), jnp.float32) if layer2
              else jax.ShapeDtypeStruct((N, 128), jnp.float32))
    return functools.partial(
        pl.kernel,
        out_type=out_ty,
        mesh=_mesh,
        scratch_types=[
            pltpu.VMEM((2, NB, EK), jnp.int32),
            pltpu.VMEM((2, NB, EK), jnp.int32),
            pltpu.VMEM((2, NB, EK, 64), jnp.float32),
            pltpu.VMEM((CH, 64), jnp.float32),
            pltpu.VMEM((CH, 64), jnp.float32),
            pltpu.VMEM((48, 64), jnp.float32),
            pltpu.VMEM((48,), jnp.float32),
            pltpu.VMEM((64,), jnp.float32),
            pltpu.VMEM((48,), jnp.int32),
            pltpu.VMEM((48,), jnp.int32),
            pltpu.VMEM_SHARED((N + 8, 64), jnp.float32),
            pltpu.VMEM_SHARED((2 * G, 64), jnp.float32),
            pltpu.SemaphoreType.DMA,
            pltpu.SemaphoreType.DMA,
            pltpu.SemaphoreType.DMA,
        ],
        compiler_params=pltpu.CompilerParams(use_tc_tiling_on_sc=False),
    )(functools.partial(_msg_body, layer2))


_msg1_kernel = _make_msg_kernel(False)
_msg2_kernel = _make_msg_kernel(True)


def _lin0_body(x_ref, w_ref, batch_ref, esrc_ref, edst_ref,
               h_ref, cnt_ref, offs_ref, dstp_ref):
    h_ref[...] = jnp.dot(x_ref[...], w_ref[...],
                         preferred_element_type=jnp.float32)
    b = batch_ref[...]
    ids = lax.broadcasted_iota(jnp.int32, (G, N), 0)
    cnt_ref[...] = jnp.sum(
        jnp.where(b[None, :] == ids, 1.0, 0.0), axis=1, keepdims=True)
    src = esrc_ref[...]
    for k in range(4):
        offs_ref[k, 0:NCHUNK, :] = src + k * N
        if NCHP > NCHUNK:
            offs_ref[k, NCHUNK:NCHP, :] = jnp.full(
                (NCHP - NCHUNK, EK), k * N, jnp.int32)
    dstp_ref[0:NCHUNK, :] = edst_ref[...]
    if NCHP > NCHUNK:
        dstp_ref[NCHUNK:NCHP, :] = jnp.full((NCHP - NCHUNK, EK), N, jnp.int32)


def _lin1_body(h_ref, deg2_ref, gs_ref, dis_ref):
    deg = deg2_ref[0, :, 0] + deg2_ref[1, :, 0] + 1.0
    dis = lax.rsqrt(deg)
    dis_ref[...] = dis
    g = h_ref[...] * dis[:, None]
    gs_ref[0:N, :] = g[:, 0:64]
    gs_ref[N:2 * N, :] = g[:, 64:128]


def _lin2_body(y_ref, w_ref, dis_ref, gs_ref):
    h = jnp.dot(y_ref[...], w_ref[...], preferred_element_type=jnp.float32)
    g = h * dis_ref[...][:, None]
    for q in range(4):
        gs_ref[q * N:(q + 1) * N, :] = g[:, q * 64:(q + 1) * 64]


def _head_body(pooled_ref, cnt_ref, w_ref, b_ref, out_ref):
    pooled = pooled_ref[...] / jnp.maximum(cnt_ref[...], 1.0)
    out_ref[...] = (
        jnp.dot(pooled, w_ref[...], preferred_element_type=jnp.float32)
        + b_ref[...][None, :])


def kernel(x, edge_index, batch, W1, b1, W2, b2, Wfc, bfc):
    zeros = jnp.zeros((N, 128), jnp.float32)
    edge3 = edge_index.reshape(2, NCHUNK, EK)

    ones8 = jnp.ones((EK, 8), jnp.float32)
    h1, cnt, offs, dstp = pl.pallas_call(
        _lin0_body,
        out_shape=[
            jax.ShapeDtypeStruct((N, D), jnp.float32),
            jax.ShapeDtypeStruct((G, 1), jnp.float32),
            jax.ShapeDtypeStruct((4, NCHP, EK), jnp.int32),
            jax.ShapeDtypeStruct((NCHP, EK), jnp.int32),
        ],
    )(x, W1, batch, edge3[0], edge3[1])

    deg2 = _deg_kernel(edge3, zeros, ones8)

    gs1, dis = pl.pallas_call(
        _lin1_body,
        out_shape=[
            jax.ShapeDtypeStruct((2 * N, 64), jnp.float32),
            jax.ShapeDtypeStruct((N,), jnp.float32),
        ],
    )(h1, deg2)

    y1 = _msg1_kernel(dstp, offs, gs1, dis, b1, batch, zeros)

    gs2 = pl.pallas_call(
        _lin2_body,
        out_shape=jax.ShapeDtypeStruct((4 * N, 64), jnp.float32),
    )(y1, W2, dis)

    pooled = _msg2_kernel(dstp, offs, gs2, dis, b2, batch, zeros)

    out = pl.pallas_call(
        _head_body,
        out_shape=jax.ShapeDtypeStruct((G, HID), jnp.float32),
    )(pooled, cnt, Wfc, bfc)
    return out
